# baseline (speedup 1.0000x reference)
import functools

import jax
import jax.numpy as jnp
from jax import lax
from jax.experimental import pallas as pl
from jax.experimental.pallas import tpu as pltpu
from jax.experimental.pallas import tpu_sc as plsc

HIDDEN = 128
VOCAB = 100000
BATCH = 1024
NATOM = 16384
NSAMP = 4
K = 5 * NSAMP
VT = 4096
VP = 102400
NTILES = VP // VT
NEG = -3.0e38
BIGI = 2 ** 30
_NCH = 256
_F = VT // _NCH
CPT = 384
NCAND = NTILES * CPT


def _sc_gather_segsum(table, wid, h, ids, zeros):
    info = plsc.get_sparse_core_info()
    nc = info.num_cores
    nw = nc * info.num_subcores
    bpw = BATCH // nw
    apw = NATOM // nw
    rpw = BATCH // info.num_subcores
    mesh = plsc.VectorSubcoreMesh(core_axis_name="c", subcore_axis_name="s")

    @functools.partial(
        pl.kernel,
        mesh=mesh,
        out_type=[
            jax.ShapeDtypeStruct((BATCH, HIDDEN), jnp.float32),
            jax.ShapeDtypeStruct((nc, BATCH, HIDDEN), jnp.float32),
        ],
        scratch_types=[
            pltpu.VMEM((bpw,), jnp.int32),
            pltpu.VMEM((bpw, HIDDEN), jnp.float32),
            pltpu.VMEM((4, 128), jnp.int32),
            pltpu.VMEM((128, HIDDEN), jnp.float32),
            pltpu.VMEM_SHARED((BATCH, HIDDEN), jnp.float32),
            pltpu.SemaphoreType.DMA,
        ],
    )
    def k(table_hbm, wid_hbm, h_hbm, ids_hbm, zeros_hbm, motif_hbm, part_hbm,
          widx_v, wrows_v, aidx_v, arows_v, shared, sem):
        c = lax.axis_index("c")
        s = lax.axis_index("s")
        w = s * nc + c
        base = w * bpw
        pltpu.sync_copy(wid_hbm.at[pl.ds(base, bpw)], widx_v)
        pltpu.async_copy(table_hbm.at[widx_v], wrows_v, sem).wait()
        pltpu.sync_copy(wrows_v, motif_hbm.at[pl.ds(base, bpw)])
        pltpu.sync_copy(zeros_hbm.at[pl.ds(s * rpw, rpw)],
                        shared.at[pl.ds(s * rpw, rpw)])
        plsc.subcore_barrier()
        abase = w * apw
        for j in range(apw // 128):
            pltpu.sync_copy(ids_hbm.at[pl.ds(abase + j * 128, 128)], aidx_v.at[j])
            pltpu.sync_copy(h_hbm.at[pl.ds(abase + j * 128, 128)], arows_v)
            pltpu.sync_copy(arows_v, shared.at[aidx_v.at[j]], add=True)
        plsc.subcore_barrier()
        pltpu.sync_copy(shared.at[pl.ds(s * rpw, rpw)],
                        part_hbm.at[c, pl.ds(s * rpw, rpw)])

    return k(table, wid, h, ids, zeros)


def _pred_body(part_ref, motif_ref, w_ref, b_ref, out_ref):
    node = part_ref[0] + part_ref[1]
    cat = jnp.concatenate([node, motif_ref[...]], axis=1)
    pre = jnp.dot(cat, w_ref[...], preferred_element_type=jnp.float32) + b_ref[...]
    out_ref[...] = jnp.maximum(pre, 0.0)


def _pred_vecs(part, motif, w, b2):
    return pl.pallas_call(
        _pred_body,
        in_specs=[
            pl.BlockSpec((2, BATCH, HIDDEN), lambda: (0, 0, 0)),
            pl.BlockSpec((BATCH, HIDDEN), lambda: (0, 0)),
            pl.BlockSpec((2 * HIDDEN, HIDDEN), lambda: (0, 0)),
            pl.BlockSpec((1, HIDDEN), lambda: (0, 0)),
        ],
        out_specs=pl.BlockSpec((BATCH, HIDDEN), lambda: (0, 0)),
        out_shape=jax.ShapeDtypeStruct((BATCH, HIDDEN), jnp.float32),
    )(part, motif, w, b2)


def _stream_body(pred_ref, wo_ref, bias_ref, cv_ref, cidx_ref, mout_ref,
                 sout_ref, lbuf_ref, m_ref, s_ref):
    i = pl.program_id(0)

    @pl.when(i == 0)
    def _():
        m_ref[...] = jnp.full_like(m_ref, NEG)
        s_ref[...] = jnp.zeros_like(s_ref)

    logits = lbuf_ref[...]

    @pl.when(i < NTILES)
    def _():
        lbuf_ref[...] = (jnp.dot(pred_ref[...], wo_ref[...],
                                 preferred_element_type=jnp.float32)
                         + bias_ref[...])

    @pl.when(i > 0)
    def _():
        t = i - 1
        m_old = m_ref[...][:, 0:1]
        s_old = s_ref[...][:, 0:1]
        tile_m = jnp.max(logits, axis=1, keepdims=True)
        m_new = jnp.maximum(m_old, tile_m)
        s_new = s_old * jnp.exp(m_old - m_new) + jnp.sum(
            jnp.exp(logits - m_new), axis=1, keepdims=True)
        m_ref[...] = jnp.broadcast_to(m_new, m_ref.shape)
        s_ref[...] = jnp.broadcast_to(s_new, s_ref.shape)
        mout_ref[...] = m_ref[...]
        sout_ref[...] = s_ref[...]

        fold = logits[:, 0:_NCH]
        for j in range(1, _F):
            fold = jnp.maximum(fold, logits[:, j * _NCH:(j + 1) * _NCH])
        ci = lax.broadcasted_iota(jnp.int32, (BATCH, _NCH), 1)
        cids = []
        for _ in range(K):
            mv = jnp.max(fold, axis=1, keepdims=True)
            eq = fold == mv
            c = jnp.min(jnp.where(eq, ci, BIGI), axis=1, keepdims=True)
            cids.append(c)
            fold = jnp.where(eq, NEG, fold)
        cid = jnp.concatenate(cids, axis=1)
        cid_lo = cid & 127
        is_lo = cid < 128
        gvs, gis = [], []
        for f in range(_F):
            lo = jnp.take_along_axis(logits[:, f * _NCH:f * _NCH + 128],
                                     cid_lo, axis=1)
            hi = jnp.take_along_axis(logits[:, f * _NCH + 128:(f + 1) * _NCH],
                                     cid_lo, axis=1)
            gvs.append(jnp.where(is_lo, lo, hi))
            gis.append(cid + f * _NCH + t * VT)
        gvs.append(jnp.full((BATCH, CPT - K * _F), NEG, jnp.float32))
        gis.append(jnp.full((BATCH, CPT - K * _F), BIGI, jnp.int32))
        cv_ref[...] = jnp.concatenate(gvs, axis=1)
        cidx_ref[...] = jnp.concatenate(gis, axis=1)


def _stream_cands(pred_vecs, wo_pad, bias_pad):
    return pl.pallas_call(
        _stream_body,
        grid=(NTILES + 1,),
        in_specs=[
            pl.BlockSpec((BATCH, HIDDEN), lambda i: (0, 0)),
            pl.BlockSpec((HIDDEN, VT), lambda i: (0, jnp.minimum(i, NTILES - 1))),
            pl.BlockSpec((1, VT), lambda i: (0, jnp.minimum(i, NTILES - 1))),
        ],
        out_specs=[
            pl.BlockSpec((BATCH, CPT), lambda i: (0, jnp.maximum(i - 1, 0))),
            pl.BlockSpec((BATCH, CPT), lambda i: (0, jnp.maximum(i - 1, 0))),
            pl.BlockSpec((BATCH, 128), lambda i: (0, 0)),
            pl.BlockSpec((BATCH, 128), lambda i: (0, 0)),
        ],
        out_shape=[
            jax.ShapeDtypeStruct((BATCH, NCAND), jnp.float32),
            jax.ShapeDtypeStruct((BATCH, NCAND), jnp.int32),
            jax.ShapeDtypeStruct((BATCH, 128), jnp.float32),
            jax.ShapeDtypeStruct((BATCH, 128), jnp.float32),
        ],
        scratch_shapes=[
            pltpu.VMEM((BATCH, VT), jnp.float32),
            pltpu.VMEM((BATCH, 128), jnp.float32),
            pltpu.VMEM((BATCH, 128), jnp.float32),
        ],
    )(pred_vecs, wo_pad, bias_pad)


_RB = 256
_NCH2 = 256
_F2 = NCAND // _NCH2


def _select_body(cv_ref, cidx_ref, m_ref, s_ref, pick_ref, preds_ref, prob_ref):
    cv = cv_ref[...]
    cidx = cidx_ref[...]
    rem = NCAND - _F2 * _NCH2
    fold = cv[:, 0:_NCH2]
    for j in range(1, _F2):
        fold = jnp.maximum(fold, cv[:, j * _NCH2:(j + 1) * _NCH2])
    if rem:
        fold = jnp.maximum(fold, jnp.concatenate(
            [cv[:, _F2 * _NCH2:], jnp.full((_RB, _NCH2 - rem), NEG, jnp.float32)],
            axis=1))
    ci = lax.broadcasted_iota(jnp.int32, (_RB, _NCH2), 1)
    cids = []
    for _ in range(K):
        mv = jnp.max(fold, axis=1, keepdims=True)
        eq = fold == mv
        c = jnp.min(jnp.where(eq, ci, BIGI), axis=1, keepdims=True)
        cids.append(c)
        fold = jnp.where(eq, NEG, fold)
    cid = jnp.concatenate(cids, axis=1)
    c_sub = cid & 127
    sel0 = cid < 128
    gvs, gis = [], []
    for f in range(_F2):
        base = f * _NCH2
        g0 = jnp.take_along_axis(cv[:, base:base + 128], c_sub, axis=1)
        g1 = jnp.take_along_axis(cv[:, base + 128:base + 256], c_sub, axis=1)
        gvs.append(jnp.where(sel0, g0, g1))
        i0 = jnp.take_along_axis(cidx[:, base:base + 128], c_sub, axis=1)
        i1 = jnp.take_along_axis(cidx[:, base + 128:base + 256], c_sub, axis=1)
        gis.append(jnp.where(sel0, i0, i1))
    if rem:
        base = _F2 * _NCH2
        g0 = jnp.take_along_axis(cv[:, base:base + rem], c_sub & (rem - 1), axis=1)
        i0 = jnp.take_along_axis(cidx[:, base:base + rem], c_sub & (rem - 1), axis=1)
        inrem = cid < rem
        gvs.append(jnp.where(inrem, g0, NEG))
        gis.append(jnp.where(inrem, i0, BIGI))
    cand = jnp.concatenate(gvs, axis=1)
    candi = jnp.concatenate(gis, axis=1)
    newv, newi = [], []
    for _ in range(K):
        mval = jnp.max(cand, axis=1, keepdims=True)
        sel = jnp.min(jnp.where(cand == mval, candi, BIGI), axis=1, keepdims=True)
        newv.append(mval)
        newi.append(sel)
        cand = jnp.where(candi == sel, NEG, cand)
    nv = jnp.concatenate(newv, axis=1)
    ni = jnp.concatenate(newi, axis=1)

    m = m_ref[...][:, 0:1]
    s = s_ref[...][:, 0:1]
    pool_prob = jnp.exp(nv - m) / s
    pres, prbs = [], []
    for j in range(NSAMP):
        pick = pick_ref[...][:, j:j + 1]
        oh = lax.broadcasted_iota(jnp.int32, (_RB, K), 1) == pick
        pres.append(jnp.sum(jnp.where(oh, ni, 0), axis=1, keepdims=True))
        prbs.append(jnp.sum(jnp.where(oh, pool_prob, 0.0), axis=1, keepdims=True))
    preds_ref[...] = jnp.concatenate(pres, axis=1)
    prob_ref[...] = jnp.concatenate(prbs, axis=1)


def _select_sample(cv, cidx, mout, sout, pick):
    nb = BATCH // _RB
    return pl.pallas_call(
        _select_body,
        grid=(nb,),
        in_specs=[
            pl.BlockSpec((_RB, NCAND), lambda i: (i, 0)),
            pl.BlockSpec((_RB, NCAND), lambda i: (i, 0)),
            pl.BlockSpec((_RB, 128), lambda i: (i, 0)),
            pl.BlockSpec((_RB, 128), lambda i: (i, 0)),
            pl.BlockSpec((_RB, NSAMP), lambda i: (i, 0)),
        ],
        out_specs=[
            pl.BlockSpec((_RB, NSAMP), lambda i: (i, 0)),
            pl.BlockSpec((_RB, NSAMP), lambda i: (i, 0)),
        ],
        out_shape=[
            jax.ShapeDtypeStruct((BATCH, NSAMP), jnp.int32),
            jax.ShapeDtypeStruct((BATCH, NSAMP), jnp.float32),
        ],
    )(cv, cidx, mout, sout, pick)


def kernel(h_ctx_focal, current_atoms_batch, current_wid, n_samples,
           emb_table, W_w, W_b, Wo_w, Wo_b):
    zeros = jnp.zeros((BATCH, HIDDEN), jnp.float32)
    motif, part = _sc_gather_segsum(emb_table, current_wid.astype(jnp.int32),
                                    h_ctx_focal,
                                    current_atoms_batch.astype(jnp.int32), zeros)
    pred_vecs = _pred_vecs(part, motif, W_w, W_b.reshape(1, HIDDEN))
    wo_pad = jnp.pad(Wo_w, ((0, 0), (0, VP - VOCAB)))
    bias_pad = jnp.concatenate(
        [Wo_b, jnp.full((VP - VOCAB,), -1e30, jnp.float32)]).reshape(1, VP)
    pick = jax.random.randint(jax.random.key(1), (BATCH, NSAMP), 0, 5 * n_samples)
    cv, cidx, mout, sout = _stream_cands(pred_vecs, wo_pad, bias_pad)
    preds2, prob2 = _select_sample(cv, cidx, mout, sout, pick)
    return preds2.reshape(-1), prob2.reshape(-1)


# speedup vs baseline: 1.0300x; 1.0300x over previous
import functools

import jax
import jax.numpy as jnp
from jax import lax
from jax.experimental import pallas as pl
from jax.experimental.pallas import tpu as pltpu
from jax.experimental.pallas import tpu_sc as plsc

HIDDEN = 128
VOCAB = 100000
BATCH = 1024
NATOM = 16384
NSAMP = 4
K = 5 * NSAMP
VT = 4096
VP = 102400
NTILES = VP // VT
NEG = -3.0e38
BIGI = 2 ** 30
_NCH = 256
_F = VT // _NCH
CPT = 384
NCAND = NTILES * CPT


def _sc_gather_segsum(table, wid, h, ids, zeros):
    info = plsc.get_sparse_core_info()
    nc = info.num_cores
    nw = nc * info.num_subcores
    bpw = BATCH // nw
    apw = NATOM // nw
    rpw = BATCH // info.num_subcores
    mesh = plsc.VectorSubcoreMesh(core_axis_name="c", subcore_axis_name="s")

    @functools.partial(
        pl.kernel,
        mesh=mesh,
        out_type=[
            jax.ShapeDtypeStruct((BATCH, HIDDEN), jnp.float32),
            jax.ShapeDtypeStruct((nc, BATCH, HIDDEN), jnp.float32),
        ],
        scratch_types=[
            pltpu.VMEM((bpw,), jnp.int32),
            pltpu.VMEM((bpw, HIDDEN), jnp.float32),
            pltpu.VMEM((4, 128), jnp.int32),
            pltpu.VMEM((128, HIDDEN), jnp.float32),
            pltpu.VMEM_SHARED((BATCH, HIDDEN), jnp.float32),
            pltpu.SemaphoreType.DMA,
        ],
    )
    def k(table_hbm, wid_hbm, h_hbm, ids_hbm, zeros_hbm, motif_hbm, part_hbm,
          widx_v, wrows_v, aidx_v, arows_v, shared, sem):
        c = lax.axis_index("c")
        s = lax.axis_index("s")
        w = s * nc + c
        base = w * bpw
        pltpu.sync_copy(wid_hbm.at[pl.ds(base, bpw)], widx_v)
        pltpu.async_copy(table_hbm.at[widx_v], wrows_v, sem).wait()
        pltpu.sync_copy(wrows_v, motif_hbm.at[pl.ds(base, bpw)])
        pltpu.sync_copy(zeros_hbm.at[pl.ds(s * rpw, rpw)],
                        shared.at[pl.ds(s * rpw, rpw)])
        plsc.subcore_barrier()
        abase = w * apw
        for j in range(apw // 128):
            pltpu.sync_copy(ids_hbm.at[pl.ds(abase + j * 128, 128)], aidx_v.at[j])
            pltpu.sync_copy(h_hbm.at[pl.ds(abase + j * 128, 128)], arows_v)
            pltpu.sync_copy(arows_v, shared.at[aidx_v.at[j]], add=True)
        plsc.subcore_barrier()
        pltpu.sync_copy(shared.at[pl.ds(s * rpw, rpw)],
                        part_hbm.at[c, pl.ds(s * rpw, rpw)])

    return k(table, wid, h, ids, zeros)


def _pred_body(part_ref, motif_ref, w_ref, b_ref, out_ref):
    node = part_ref[0] + part_ref[1]
    cat = jnp.concatenate([node, motif_ref[...]], axis=1)
    pre = jnp.dot(cat, w_ref[...], preferred_element_type=jnp.float32) + b_ref[...]
    out_ref[...] = jnp.maximum(pre, 0.0)


def _pred_vecs(part, motif, w, b2):
    return pl.pallas_call(
        _pred_body,
        in_specs=[
            pl.BlockSpec((2, BATCH, HIDDEN), lambda: (0, 0, 0)),
            pl.BlockSpec((BATCH, HIDDEN), lambda: (0, 0)),
            pl.BlockSpec((2 * HIDDEN, HIDDEN), lambda: (0, 0)),
            pl.BlockSpec((1, HIDDEN), lambda: (0, 0)),
        ],
        out_specs=pl.BlockSpec((BATCH, HIDDEN), lambda: (0, 0)),
        out_shape=jax.ShapeDtypeStruct((BATCH, HIDDEN), jnp.float32),
    )(part, motif, w, b2)


def _stream_body(pred_ref, wo_ref, bias_ref, cv_ref, cidx_ref, mout_ref,
                 sout_ref, m_ref, s_ref):
    i = pl.program_id(0)
    logits = (jnp.dot(pred_ref[...], wo_ref[...], preferred_element_type=jnp.float32)
              + bias_ref[...])

    @pl.when(i == 0)
    def _():
        m_ref[...] = jnp.full_like(m_ref, NEG)
        s_ref[...] = jnp.zeros_like(s_ref)

    m_old = m_ref[...][:, 0:1]
    s_old = s_ref[...][:, 0:1]
    tile_m = jnp.max(logits, axis=1, keepdims=True)
    m_new = jnp.maximum(m_old, tile_m)
    s_new = s_old * jnp.exp(m_old - m_new) + jnp.sum(
        jnp.exp(logits - m_new), axis=1, keepdims=True)
    m_ref[...] = jnp.broadcast_to(m_new, m_ref.shape)
    s_ref[...] = jnp.broadcast_to(s_new, s_ref.shape)
    mout_ref[...] = m_ref[...]
    sout_ref[...] = s_ref[...]

    fold = logits[:, 0:_NCH]
    for j in range(1, _F):
        fold = jnp.maximum(fold, logits[:, j * _NCH:(j + 1) * _NCH])
    ci = lax.broadcasted_iota(jnp.int32, (BATCH, _NCH), 1)
    cids = []
    for _ in range(K):
        mv = jnp.max(fold, axis=1, keepdims=True)
        eq = fold == mv
        c = jnp.min(jnp.where(eq, ci, BIGI), axis=1, keepdims=True)
        cids.append(c)
        fold = jnp.where(eq, NEG, fold)
    cid = jnp.concatenate(cids, axis=1)
    cid_lo = cid & 127
    is_lo = cid < 128
    gvs, gis = [], []
    for f in range(_F):
        lo = jnp.take_along_axis(logits[:, f * _NCH:f * _NCH + 128], cid_lo, axis=1)
        hi = jnp.take_along_axis(logits[:, f * _NCH + 128:(f + 1) * _NCH], cid_lo, axis=1)
        gvs.append(jnp.where(is_lo, lo, hi))
        gis.append(cid + f * _NCH + i * VT)
    gvs.append(jnp.full((BATCH, CPT - K * _F), NEG, jnp.float32))
    gis.append(jnp.full((BATCH, CPT - K * _F), BIGI, jnp.int32))
    cv_ref[...] = jnp.concatenate(gvs, axis=1)
    cidx_ref[...] = jnp.concatenate(gis, axis=1)


def _stream_cands(pred_vecs, wo_pad, bias_pad):
    return pl.pallas_call(
        _stream_body,
        grid=(NTILES,),
        in_specs=[
            pl.BlockSpec((BATCH, HIDDEN), lambda i: (0, 0)),
            pl.BlockSpec((HIDDEN, VT), lambda i: (0, i)),
            pl.BlockSpec((1, VT), lambda i: (0, i)),
        ],
        out_specs=[
            pl.BlockSpec((BATCH, CPT), lambda i: (0, i)),
            pl.BlockSpec((BATCH, CPT), lambda i: (0, i)),
            pl.BlockSpec((BATCH, 128), lambda i: (0, 0)),
            pl.BlockSpec((BATCH, 128), lambda i: (0, 0)),
        ],
        out_shape=[
            jax.ShapeDtypeStruct((BATCH, NCAND), jnp.float32),
            jax.ShapeDtypeStruct((BATCH, NCAND), jnp.int32),
            jax.ShapeDtypeStruct((BATCH, 128), jnp.float32),
            jax.ShapeDtypeStruct((BATCH, 128), jnp.float32),
        ],
        scratch_shapes=[
            pltpu.VMEM((BATCH, 128), jnp.float32),
            pltpu.VMEM((BATCH, 128), jnp.float32),
        ],
    )(pred_vecs, wo_pad, bias_pad)


_RB = 256
_NCH2 = 256
_F2 = NCAND // _NCH2


def _select_body(cv_ref, cidx_ref, m_ref, s_ref, pick_ref, preds_ref, prob_ref):
    cv = cv_ref[...]
    cidx = cidx_ref[...]
    rem = NCAND - _F2 * _NCH2
    fold = cv[:, 0:_NCH2]
    for j in range(1, _F2):
        fold = jnp.maximum(fold, cv[:, j * _NCH2:(j + 1) * _NCH2])
    if rem:
        fold = jnp.maximum(fold, jnp.concatenate(
            [cv[:, _F2 * _NCH2:], jnp.full((_RB, _NCH2 - rem), NEG, jnp.float32)],
            axis=1))
    ci = lax.broadcasted_iota(jnp.int32, (_RB, _NCH2), 1)
    cids = []
    for _ in range(K):
        mv = jnp.max(fold, axis=1, keepdims=True)
        eq = fold == mv
        c = jnp.min(jnp.where(eq, ci, BIGI), axis=1, keepdims=True)
        cids.append(c)
        fold = jnp.where(eq, NEG, fold)
    cid = jnp.concatenate(cids, axis=1)
    c_sub = cid & 127
    sel0 = cid < 128
    gvs, gis = [], []
    for f in range(_F2):
        base = f * _NCH2
        g0 = jnp.take_along_axis(cv[:, base:base + 128], c_sub, axis=1)
        g1 = jnp.take_along_axis(cv[:, base + 128:base + 256], c_sub, axis=1)
        gvs.append(jnp.where(sel0, g0, g1))
        i0 = jnp.take_along_axis(cidx[:, base:base + 128], c_sub, axis=1)
        i1 = jnp.take_along_axis(cidx[:, base + 128:base + 256], c_sub, axis=1)
        gis.append(jnp.where(sel0, i0, i1))
    if rem:
        base = _F2 * _NCH2
        g0 = jnp.take_along_axis(cv[:, base:base + rem], c_sub & (rem - 1), axis=1)
        i0 = jnp.take_along_axis(cidx[:, base:base + rem], c_sub & (rem - 1), axis=1)
        inrem = cid < rem
        gvs.append(jnp.where(inrem, g0, NEG))
        gis.append(jnp.where(inrem, i0, BIGI))
    cand = jnp.concatenate(gvs, axis=1)
    candi = jnp.concatenate(gis, axis=1)
    newv, newi = [], []
    for _ in range(K):
        mval = jnp.max(cand, axis=1, keepdims=True)
        sel = jnp.min(jnp.where(cand == mval, candi, BIGI), axis=1, keepdims=True)
        newv.append(mval)
        newi.append(sel)
        cand = jnp.where(candi == sel, NEG, cand)
    nv = jnp.concatenate(newv, axis=1)
    ni = jnp.concatenate(newi, axis=1)

    m = m_ref[...][:, 0:1]
    s = s_ref[...][:, 0:1]
    pool_prob = jnp.exp(nv - m) / s
    pres, prbs = [], []
    for j in range(NSAMP):
        pick = pick_ref[...][:, j:j + 1]
        oh = lax.broadcasted_iota(jnp.int32, (_RB, K), 1) == pick
        pres.append(jnp.sum(jnp.where(oh, ni, 0), axis=1, keepdims=True))
        prbs.append(jnp.sum(jnp.where(oh, pool_prob, 0.0), axis=1, keepdims=True))
    preds_ref[...] = jnp.concatenate(pres, axis=1)
    prob_ref[...] = jnp.concatenate(prbs, axis=1)


def _select_sample(cv, cidx, mout, sout, pick):
    nb = BATCH // _RB
    return pl.pallas_call(
        _select_body,
        grid=(nb,),
        in_specs=[
            pl.BlockSpec((_RB, NCAND), lambda i: (i, 0)),
            pl.BlockSpec((_RB, NCAND), lambda i: (i, 0)),
            pl.BlockSpec((_RB, 128), lambda i: (i, 0)),
            pl.BlockSpec((_RB, 128), lambda i: (i, 0)),
            pl.BlockSpec((_RB, NSAMP), lambda i: (i, 0)),
        ],
        out_specs=[
            pl.BlockSpec((_RB, NSAMP), lambda i: (i, 0)),
            pl.BlockSpec((_RB, NSAMP), lambda i: (i, 0)),
        ],
        out_shape=[
            jax.ShapeDtypeStruct((BATCH, NSAMP), jnp.int32),
            jax.ShapeDtypeStruct((BATCH, NSAMP), jnp.float32),
        ],
    )(cv, cidx, mout, sout, pick)


def kernel(h_ctx_focal, current_atoms_batch, current_wid, n_samples,
           emb_table, W_w, W_b, Wo_w, Wo_b):
    zeros = jnp.zeros((BATCH, HIDDEN), jnp.float32)
    motif, part = _sc_gather_segsum(emb_table, current_wid.astype(jnp.int32),
                                    h_ctx_focal,
                                    current_atoms_batch.astype(jnp.int32), zeros)
    pred_vecs = _pred_vecs(part, motif, W_w, W_b.reshape(1, HIDDEN))
    wo_pad = jnp.pad(Wo_w, ((0, 0), (0, VP - VOCAB)))
    bias_pad = jnp.concatenate(
        [Wo_b, jnp.full((VP - VOCAB,), -1e30, jnp.float32)]).reshape(1, VP)
    pick = jax.random.randint(jax.random.key(1), (BATCH, NSAMP), 0, 5 * n_samples)
    cv, cidx, mout, sout = _stream_cands(pred_vecs, wo_pad, bias_pad)
    preds2, prob2 = _select_sample(cv, cidx, mout, sout, pick)
    return preds2.reshape(-1), prob2.reshape(-1)
